# baseline (speedup 1.0000x reference)
import jax
import jax.numpy as jnp
from jax.experimental import pallas as pl
from jax.experimental.pallas import tpu as pltpu
from jax.experimental.pallas import tpu_sc as plsc

M = 100000
D = 32
B = 16384
K = 1000
LR = 0.01
LIMIT = 1.0

NC = 2
NS = 16
ROW_BLK = 1024
DBLK = ROW_BLK // 4
NBLK = (M + ROW_BLK - 1) // ROW_BLK
MB = NBLK * ROW_BLK
MD = MB // 4
NPAD_ROWS = NBLK * 8

HALF_B = MB // NC
RPT = HALF_B // NS
GPT = B // NS
NCHUNK = GPT // 128


def _to_dense(y):
    return jnp.concatenate(
        [y[b * DBLK:(b + 1) * DBLK, :] for b in range(4)], axis=1)


def _to_rows(x):
    return jnp.concatenate(
        [x[:, b * D:(b + 1) * D] for b in range(4)], axis=0)


def _clip_body(x_ref, o_ref):
    x = x_ref[...]
    n = jnp.sqrt(jnp.sum(x * x, axis=1, keepdims=True))
    scale = jnp.where(n > LIMIT, LIMIT / jnp.maximum(n, 1e-12), 1.0)
    o_ref[...] = _to_dense(x * scale)


def _repack_body(x_ref, o_ref):
    o_ref[...] = _to_dense(x_ref[...])


def _sc_bank_body(bank_in, g3d, items_hbm, bank_out, acc, g_buf, idx_buf,
                  tgt_buf):
    c = jax.lax.axis_index("c")
    s = jax.lax.axis_index("s")
    base_row = c * HALF_B + s * RPT
    pltpu.sync_copy(bank_in.at[pl.ds(base_row, RPT)],
                    acc.at[pl.ds(s * RPT, RPT)])
    for j in range(NCHUNK):
        pltpu.sync_copy(items_hbm.at[pl.ds(s * GPT + j * 128, 128)],
                        idx_buf.at[j])
    lo = c * HALF_B
    for cc in range(NCHUNK):
        for q in range(8):
            rbv = cc * 128 + q * 16 + jax.lax.broadcasted_iota(
                jnp.int32, (16,), 0)
            p = (rbv & 3) * DBLK + (rbv >> 2)
            v = plsc.load_gather(idx_buf, [p >> 7, p & 127])
            loc = v & (ROW_BLK - 1)
            rb_t = (v - loc) + ((loc & (DBLK - 1)) << 2) + (loc >> 8)
            lrb = rb_t - lo
            oob = (lrb < 0) | (lrb >= HALF_B)
            tgt_buf[cc, pl.ds(q * 16, 16)] = jnp.where(oob, HALF_B, lrb)
    plsc.subcore_barrier()
    for h in range(2):
        pltpu.sync_copy(g3d.at[s, h], g_buf)
        for j in range(NCHUNK // 2):
            pltpu.sync_copy(g_buf.at[j],
                            acc.at[tgt_buf.at[h * (NCHUNK // 2) + j]],
                            add=True)
    plsc.subcore_barrier()
    pltpu.sync_copy(acc.at[pl.ds(s * RPT, RPT)],
                    bank_out.at[pl.ds(base_row, RPT)])


def _mega_body(bank_ref, emb_ref, o_ref, n2s_ref, vt_ref):
    i = pl.program_id(0)

    @pl.when(i < NBLK)
    def _phase1():
        bk = _to_rows(bank_ref[...])
        x3 = bk.reshape(8, 128, D)
        n2 = jnp.sum(x3 * x3, axis=2)
        rid = (i * ROW_BLK
               + jax.lax.broadcasted_iota(jnp.int32, (8, 128), 0) * 128
               + jax.lax.broadcasted_iota(jnp.int32, (8, 128), 1))
        n2s_ref[pl.ds(i * 8, 8), :] = jnp.where(rid < M, n2, 0.0)

    @pl.when(i == NBLK)
    def _phase2():
        x = n2s_ref[...]
        bits = jax.lax.bitcast_convert_type(x, jnp.int32)

        def step(p, u):
            shift = 28 - 4 * p
            dstar = jnp.int32(0)
            for d in range(1, 16):
                cand = jnp.bitwise_or(u, jnp.left_shift(jnp.int32(d), shift))
                ok = jnp.logical_and(
                    cand > 0,
                    jnp.sum((bits >= cand).astype(jnp.int32)) >= K)
                dstar = dstar + ok.astype(jnp.int32)
            return jnp.bitwise_or(u, jnp.left_shift(dstar, shift))

        u = jax.lax.fori_loop(0, 8, step, jnp.int32(0))
        v = jax.lax.bitcast_convert_type(u, jnp.float32)
        tsum = jnp.sum(jnp.where(x >= v, jnp.sqrt(x), 0.0))
        vt_ref[0] = v
        vt_ref[1] = tsum / K

    @pl.when(i > NBLK)
    def _phase3():
        bk = _to_rows(bank_ref[...])
        e = emb_ref[...]
        v = vt_ref[0]
        t = vt_ref[1]
        n2 = jnp.sum(bk * bk, axis=1, keepdims=True)
        bn = jnp.sqrt(n2)
        scale = jnp.where(bn > t, t / jnp.maximum(bn, 1e-12), 1.0)
        upd = jnp.where(n2 >= v, bk * scale, 0.0)
        o_ref[...] = e - LR * upd


def kernel(items_emb, item_grad_bank, items, items_emb_grad):
    f32 = jnp.float32
    g_dense = pl.pallas_call(
        _clip_body,
        grid=(B // ROW_BLK,),
        in_specs=[pl.BlockSpec((ROW_BLK, D), lambda i: (i, 0))],
        out_specs=pl.BlockSpec((DBLK, 128), lambda i: (i, 0)),
        out_shape=jax.ShapeDtypeStruct((B // 4, 128), f32),
    )(items_emb_grad)

    igb_dense = pl.pallas_call(
        _repack_body,
        grid=(NBLK,),
        in_specs=[pl.BlockSpec((ROW_BLK, D), lambda i: (i, 0))],
        out_specs=pl.BlockSpec((DBLK, 128), lambda i: (i, 0)),
        out_shape=jax.ShapeDtypeStruct((MD, 128), f32),
    )(item_grad_bank)

    g3d = g_dense.reshape(NS, 2, NCHUNK // 2, 128, D)
    mesh = plsc.VectorSubcoreMesh(core_axis_name="c", subcore_axis_name="s")
    bank = pl.kernel(
        _sc_bank_body,
        out_type=jax.ShapeDtypeStruct((MB, D), f32),
        mesh=mesh,
        scratch_types=[
            pltpu.VMEM_SHARED((HALF_B + 8, D), f32),
            pltpu.VMEM((NCHUNK // 2, 128, D), f32),
            pltpu.VMEM((NCHUNK, 128), jnp.int32),
            pltpu.VMEM((NCHUNK, 128), jnp.int32),
        ],
        compiler_params=pltpu.CompilerParams(
            use_tc_tiling_on_sc=False, needs_layout_passes=False),
    )(igb_dense.reshape(MB, D), g3d, items)
    bank_dense = bank.reshape(MD, 128)

    def bank_idx(i):
        return (jnp.where(i < NBLK, i, jnp.where(i == NBLK, 0, i - NBLK - 1)), 0)

    def emb_idx(i):
        return (jnp.where(i <= NBLK, 0, i - NBLK - 1), 0)

    out = pl.pallas_call(
        _mega_body,
        grid=(2 * NBLK + 1,),
        in_specs=[
            pl.BlockSpec((DBLK, 128), bank_idx),
            pl.BlockSpec((ROW_BLK, D), emb_idx),
        ],
        out_specs=pl.BlockSpec((ROW_BLK, D), emb_idx),
        out_shape=jax.ShapeDtypeStruct((M, D), f32),
        scratch_shapes=[
            pltpu.VMEM((NPAD_ROWS, 128), f32),
            pltpu.SMEM((2,), f32),
        ],
    )(bank_dense, items_emb)
    return out


# speedup vs baseline: 1.3630x; 1.3630x over previous
import jax
import jax.numpy as jnp
from jax.experimental import pallas as pl
from jax.experimental.pallas import tpu as pltpu
from jax.experimental.pallas import tpu_sc as plsc

M = 100000
D = 32
B = 16384
K = 1000
LR = 0.01
LIMIT = 1.0

NC = 2
NS = 16
ROW_BLK = 1024
DBLK = ROW_BLK // 4
NBLK = (M + ROW_BLK - 1) // ROW_BLK
MB = NBLK * ROW_BLK
MD = MB // 4
NPAD_ROWS = NBLK * 8

HALF_B = MB // NC
RPT = HALF_B // NS
GPT = B // NS
NCHUNK = GPT // 128


def _to_dense(y):
    return jnp.concatenate(
        [y[b * DBLK:(b + 1) * DBLK, :] for b in range(4)], axis=1)


def _to_rows(x):
    return jnp.concatenate(
        [x[:, b * D:(b + 1) * D] for b in range(4)], axis=0)


def _clip_body(x_ref, o_ref):
    x = x_ref[...]
    n = jnp.sqrt(jnp.sum(x * x, axis=0, keepdims=True))
    scale = jnp.where(n > LIMIT, LIMIT / jnp.maximum(n, 1e-12), 1.0)
    o_ref[...] = _to_dense(jnp.transpose(x * scale))


def _repack_body(x_ref, o_ref):
    o_ref[...] = _to_dense(jnp.transpose(x_ref[...]))


def _sc_bank_body(bank_in, g3d, items_hbm, bank_out, acc, g_buf, idx_buf,
                  tgt_buf):
    c = jax.lax.axis_index("c")
    s = jax.lax.axis_index("s")
    base_row = c * HALF_B + s * RPT
    pltpu.sync_copy(bank_in.at[pl.ds(base_row, RPT)],
                    acc.at[pl.ds(s * RPT, RPT)])
    for j in range(NCHUNK):
        pltpu.sync_copy(items_hbm.at[pl.ds(s * GPT + j * 128, 128)],
                        idx_buf.at[j])
    lo = c * HALF_B
    for cc in range(NCHUNK):
        for q in range(8):
            rbv = cc * 128 + q * 16 + jax.lax.broadcasted_iota(
                jnp.int32, (16,), 0)
            p = (rbv & 3) * DBLK + (rbv >> 2)
            v = plsc.load_gather(idx_buf, [p >> 7, p & 127])
            loc = v & (ROW_BLK - 1)
            rb_t = (v - loc) + ((loc & (DBLK - 1)) << 2) + (loc >> 8)
            lrb = rb_t - lo
            oob = (lrb < 0) | (lrb >= HALF_B)
            tgt_buf[cc, pl.ds(q * 16, 16)] = jnp.where(oob, HALF_B, lrb)
    plsc.subcore_barrier()
    for h in range(2):
        pltpu.sync_copy(g3d.at[s, h], g_buf)
        for j in range(NCHUNK // 2):
            pltpu.sync_copy(g_buf.at[j],
                            acc.at[tgt_buf.at[h * (NCHUNK // 2) + j]],
                            add=True)
    plsc.subcore_barrier()
    pltpu.sync_copy(acc.at[pl.ds(s * RPT, RPT)],
                    bank_out.at[pl.ds(base_row, RPT)])


def _m1_body(bank_ref, v_ref, t_ref, n2s_ref):
    i = pl.program_id(0)

    @pl.when(i < NBLK)
    def _phase1():
        bk_t = jnp.transpose(_to_rows(bank_ref[...]))
        n2 = jnp.sum(bk_t * bk_t, axis=0, keepdims=True)
        rid = (i * ROW_BLK
               + jax.lax.broadcasted_iota(jnp.int32, (1, ROW_BLK), 1))
        n2s_ref[pl.ds(i, 1), :] = jnp.where(rid < M, n2, 0.0)

    @pl.when(i == NBLK)
    def _phase2():
        x = n2s_ref[...]
        bits = jax.lax.bitcast_convert_type(x, jnp.int32)

        def step(p, u):
            shift = 28 - 4 * p
            dstar = jnp.int32(0)
            for d in range(1, 16):
                cand = jnp.bitwise_or(u, jnp.left_shift(jnp.int32(d), shift))
                ok = jnp.logical_and(
                    cand > 0,
                    jnp.sum((bits >= cand).astype(jnp.int32)) >= K)
                dstar = dstar + ok.astype(jnp.int32)
            return jnp.bitwise_or(u, jnp.left_shift(dstar, shift))

        u = jax.lax.fori_loop(0, 8, step, jnp.int32(0))
        v = jax.lax.bitcast_convert_type(u, jnp.float32)
        tsum = jnp.sum(jnp.where(x >= v, jnp.sqrt(x), 0.0))
        v_ref[...] = jnp.full((1, 1), v, jnp.float32)
        t_ref[...] = jnp.full((1, 1), tsum / K, jnp.float32)


def _m2_body(bank_ref, e_ref, v_ref, t_ref, o_ref):
    bk_t = jnp.transpose(_to_rows(bank_ref[...]))
    e = e_ref[...]
    v = v_ref[0, 0]
    t = t_ref[0, 0]
    n2 = jnp.sum(bk_t * bk_t, axis=0, keepdims=True)
    bn = jnp.sqrt(n2)
    scale = jnp.where(bn > t, t / jnp.maximum(bn, 1e-12), 1.0)
    factor = jnp.where(n2 >= v, scale, 0.0)
    o_ref[...] = e - LR * (bk_t * factor)


def kernel(items_emb, item_grad_bank, items, items_emb_grad):
    f32 = jnp.float32
    emb_t = jnp.transpose(items_emb)
    igb_t = jnp.transpose(item_grad_bank)
    grad_t = jnp.transpose(items_emb_grad)

    g_dense = pl.pallas_call(
        _clip_body,
        grid=(B // ROW_BLK,),
        in_specs=[pl.BlockSpec((D, ROW_BLK), lambda i: (0, i))],
        out_specs=pl.BlockSpec((DBLK, 128), lambda i: (i, 0)),
        out_shape=jax.ShapeDtypeStruct((B // 4, 128), f32),
    )(grad_t)

    igb_dense = pl.pallas_call(
        _repack_body,
        grid=(NBLK,),
        in_specs=[pl.BlockSpec((D, ROW_BLK), lambda i: (0, i))],
        out_specs=pl.BlockSpec((DBLK, 128), lambda i: (i, 0)),
        out_shape=jax.ShapeDtypeStruct((MD, 128), f32),
    )(igb_t)

    g3d = g_dense.reshape(NS, 2, NCHUNK // 2, 128, D)
    mesh = plsc.VectorSubcoreMesh(core_axis_name="c", subcore_axis_name="s")
    bank = pl.kernel(
        _sc_bank_body,
        out_type=jax.ShapeDtypeStruct((MB, D), f32),
        mesh=mesh,
        scratch_types=[
            pltpu.VMEM_SHARED((HALF_B + 8, D), f32),
            pltpu.VMEM((NCHUNK // 2, 128, D), f32),
            pltpu.VMEM((NCHUNK, 128), jnp.int32),
            pltpu.VMEM((NCHUNK, 128), jnp.int32),
        ],
        compiler_params=pltpu.CompilerParams(
            use_tc_tiling_on_sc=False, needs_layout_passes=False),
    )(igb_dense.reshape(MB, D), g3d, items)
    bank_dense = bank.reshape(MD, 128)

    v, t = pl.pallas_call(
        _m1_body,
        grid=(NBLK + 1,),
        in_specs=[pl.BlockSpec((DBLK, 128),
                               lambda i: (jnp.where(i < NBLK, i, 0), 0))],
        out_specs=[pl.BlockSpec((1, 1), lambda i: (0, 0)),
                   pl.BlockSpec((1, 1), lambda i: (0, 0))],
        out_shape=[jax.ShapeDtypeStruct((1, 1), f32),
                   jax.ShapeDtypeStruct((1, 1), f32)],
        scratch_shapes=[pltpu.VMEM((NBLK, ROW_BLK), f32)],
    )(bank_dense)

    out_t = pl.pallas_call(
        _m2_body,
        grid=(NBLK,),
        in_specs=[
            pl.BlockSpec((DBLK, 128), lambda i: (i, 0)),
            pl.BlockSpec((D, ROW_BLK), lambda i: (0, i)),
            pl.BlockSpec(memory_space=pltpu.MemorySpace.SMEM),
            pl.BlockSpec(memory_space=pltpu.MemorySpace.SMEM),
        ],
        out_specs=pl.BlockSpec((D, ROW_BLK), lambda i: (0, i)),
        out_shape=jax.ShapeDtypeStruct((D, M), f32),
    )(bank_dense, emb_t, v, t)
    return jnp.transpose(out_t)


# speedup vs baseline: 1.7783x; 1.3047x over previous
import jax
import jax.numpy as jnp
from jax.experimental import pallas as pl
from jax.experimental.pallas import tpu as pltpu
from jax.experimental.pallas import tpu_sc as plsc

M = 100000
D = 32
B = 16384
K = 1000
LR = 0.01
LIMIT = 1.0

NC = 2
NS = 16
ROW_BLK = 1024
DBLK = ROW_BLK // 4
NBLK = (M + ROW_BLK - 1) // ROW_BLK
MB = NBLK * ROW_BLK
MD = MB // 4
NPAD_ROWS = NBLK * 8

HALF_B = MB // NC
RPT = HALF_B // NS
GPT = B // NS
NCHUNK = GPT // 128
ZROWS = RPT // 16


def _to_dense(y):
    return jnp.concatenate(
        [y[b * DBLK:(b + 1) * DBLK, :] for b in range(4)], axis=1)


def _to_rows(x):
    return jnp.concatenate(
        [x[:, b * D:(b + 1) * D] for b in range(4)], axis=0)


def _clip_body(x_ref, o_ref):
    x = x_ref[...]
    n = jnp.sqrt(jnp.sum(x * x, axis=0, keepdims=True))
    scale = jnp.where(n > LIMIT, LIMIT / jnp.maximum(n, 1e-12), 1.0)
    o_ref[...] = _to_dense(jnp.transpose(x * scale))


def _sc_bank_body(g3d, items_hbm, bg_out, acc, g_buf, idx_buf,
                  tgt_buf, zbuf):
    c = jax.lax.axis_index("c")
    s = jax.lax.axis_index("s")
    base_row = c * HALF_B + s * RPT
    zv = jnp.zeros((16,), jnp.float32)

    def _zrow(j, carry):
        zbuf[j, pl.ds(0, 16)] = zv
        zbuf[j, pl.ds(16, 16)] = zv
        return carry

    jax.lax.fori_loop(0, ZROWS, _zrow, jnp.int32(0))
    for k in range(RPT // ZROWS):
        pltpu.sync_copy(zbuf, acc.at[pl.ds(s * RPT + k * ZROWS, ZROWS)])
    for j in range(NCHUNK):
        pltpu.sync_copy(items_hbm.at[pl.ds(s * GPT + j * 128, 128)],
                        idx_buf.at[j])
    lo = c * HALF_B
    for cc in range(NCHUNK):
        for q in range(8):
            rbv = cc * 128 + q * 16 + jax.lax.broadcasted_iota(
                jnp.int32, (16,), 0)
            p = (rbv & 3) * DBLK + (rbv >> 2)
            v = plsc.load_gather(idx_buf, [p >> 7, p & 127])
            loc = v & (ROW_BLK - 1)
            rb_t = (v - loc) + ((loc & (DBLK - 1)) << 2) + (loc >> 8)
            lrb = rb_t - lo
            oob = (lrb < 0) | (lrb >= HALF_B)
            tgt_buf[cc, pl.ds(q * 16, 16)] = jnp.where(oob, HALF_B, lrb)
    plsc.subcore_barrier()
    for h in range(2):
        pltpu.sync_copy(g3d.at[s, h], g_buf)
        for j in range(NCHUNK // 2):
            pltpu.sync_copy(g_buf.at[j],
                            acc.at[tgt_buf.at[h * (NCHUNK // 2) + j]],
                            add=True)
    plsc.subcore_barrier()
    pltpu.sync_copy(acc.at[pl.ds(s * RPT, RPT)],
                    bg_out.at[pl.ds(base_row, RPT)])


def _m1_body(bg_ref, igb_ref, v_ref, t_ref, n2s_ref):
    i = pl.program_id(0)

    @pl.when(i < NBLK)
    def _phase1():
        bk_t = (jnp.transpose(_to_rows(bg_ref[...]))
                + igb_ref[...])
        n2 = jnp.sum(bk_t * bk_t, axis=0, keepdims=True)
        rid = (i * ROW_BLK
               + jax.lax.broadcasted_iota(jnp.int32, (1, ROW_BLK), 1))
        n2s_ref[pl.ds(i, 1), :] = jnp.where(rid < M, n2, 0.0)

    @pl.when(i == NBLK)
    def _phase2():
        x = n2s_ref[...]
        bits = jax.lax.bitcast_convert_type(x, jnp.int32)

        def step(p, u):
            shift = 28 - 4 * p
            dstar = jnp.int32(0)
            for d in range(1, 16):
                cand = jnp.bitwise_or(u, jnp.left_shift(jnp.int32(d), shift))
                ok = jnp.logical_and(
                    cand > 0,
                    jnp.sum((bits >= cand).astype(jnp.int32)) >= K)
                dstar = dstar + ok.astype(jnp.int32)
            return jnp.bitwise_or(u, jnp.left_shift(dstar, shift))

        u = jax.lax.fori_loop(0, 8, step, jnp.int32(0))
        v = jax.lax.bitcast_convert_type(u, jnp.float32)
        tsum = jnp.sum(jnp.where(x >= v, jnp.sqrt(x), 0.0))
        v_ref[...] = jnp.full((1, 1), v, jnp.float32)
        t_ref[...] = jnp.full((1, 1), tsum / K, jnp.float32)


def _m2_body(bg_ref, igb_ref, e_ref, v_ref, t_ref, o_ref):
    bk_t = (jnp.transpose(_to_rows(bg_ref[...]))
            + igb_ref[...])
    e = e_ref[...]
    v = v_ref[0, 0]
    t = t_ref[0, 0]
    n2 = jnp.sum(bk_t * bk_t, axis=0, keepdims=True)
    bn = jnp.sqrt(n2)
    scale = jnp.where(bn > t, t / jnp.maximum(bn, 1e-12), 1.0)
    factor = jnp.where(n2 >= v, scale, 0.0)
    o_ref[...] = e - LR * (bk_t * factor)


def kernel(items_emb, item_grad_bank, items, items_emb_grad):
    f32 = jnp.float32
    emb_t = jnp.transpose(items_emb)
    igb_t = jnp.transpose(item_grad_bank)
    grad_t = jnp.transpose(items_emb_grad)

    g_dense = pl.pallas_call(
        _clip_body,
        grid=(B // ROW_BLK,),
        in_specs=[pl.BlockSpec((D, ROW_BLK), lambda i: (0, i))],
        out_specs=pl.BlockSpec((DBLK, 128), lambda i: (i, 0)),
        out_shape=jax.ShapeDtypeStruct((B // 4, 128), f32),
    )(grad_t)

    g3d = g_dense.reshape(NS, 2, NCHUNK // 2, 128, D)
    mesh = plsc.VectorSubcoreMesh(core_axis_name="c", subcore_axis_name="s")
    bg = pl.kernel(
        _sc_bank_body,
        out_type=jax.ShapeDtypeStruct((MB, D), f32),
        mesh=mesh,
        scratch_types=[
            pltpu.VMEM_SHARED((HALF_B + 8, D), f32),
            pltpu.VMEM((NCHUNK // 2, 128, D), f32),
            pltpu.VMEM((NCHUNK, 128), jnp.int32),
            pltpu.VMEM((NCHUNK, 128), jnp.int32),
            pltpu.VMEM((ZROWS, D), f32),
        ],
        compiler_params=pltpu.CompilerParams(
            use_tc_tiling_on_sc=False, needs_layout_passes=False),
    )(g3d, items)
    bg_dense = bg.reshape(MD, 128)

    v, t = pl.pallas_call(
        _m1_body,
        grid=(NBLK + 1,),
        in_specs=[
            pl.BlockSpec((DBLK, 128),
                         lambda i: (jnp.where(i < NBLK, i, 0), 0)),
            pl.BlockSpec((D, ROW_BLK),
                         lambda i: (0, jnp.where(i < NBLK, i, 0))),
        ],
        out_specs=[pl.BlockSpec((1, 1), lambda i: (0, 0)),
                   pl.BlockSpec((1, 1), lambda i: (0, 0))],
        out_shape=[jax.ShapeDtypeStruct((1, 1), f32),
                   jax.ShapeDtypeStruct((1, 1), f32)],
        scratch_shapes=[pltpu.VMEM((NBLK, ROW_BLK), f32)],
    )(bg_dense, igb_t)

    out_t = pl.pallas_call(
        _m2_body,
        grid=(NBLK,),
        in_specs=[
            pl.BlockSpec((DBLK, 128), lambda i: (i, 0)),
            pl.BlockSpec((D, ROW_BLK), lambda i: (0, i)),
            pl.BlockSpec((D, ROW_BLK), lambda i: (0, i)),
            pl.BlockSpec(memory_space=pltpu.MemorySpace.SMEM),
            pl.BlockSpec(memory_space=pltpu.MemorySpace.SMEM),
        ],
        out_specs=pl.BlockSpec((D, ROW_BLK), lambda i: (0, i)),
        out_shape=jax.ShapeDtypeStruct((D, M), f32),
    )(bg_dense, igb_t, emb_t, v, t)
    return jnp.transpose(out_t)


# speedup vs baseline: 2.3784x; 1.3375x over previous
import jax
import jax.numpy as jnp
from jax.experimental import pallas as pl
from jax.experimental.pallas import tpu as pltpu
from jax.experimental.pallas import tpu_sc as plsc

M = 100000
D = 32
B = 16384
K = 1000
LR = 0.01
LIMIT = 1.0

NC = 2
NS = 16
ROW_BLK = 2048
DBLK = ROW_BLK // 4
DSH = 9
NBLK = (M + ROW_BLK - 1) // ROW_BLK
MB = NBLK * ROW_BLK
MD = MB // 4
NPAD_ROWS = NBLK * 8

HALF_B = MB // NC
RPT = HALF_B // NS
GPT = B // NS
NCHUNK = GPT // 128
ZROWS = RPT // 16


def _to_dense(y):
    return jnp.concatenate(
        [y[b * DBLK:(b + 1) * DBLK, :] for b in range(4)], axis=1)


def _to_rows(x):
    return jnp.concatenate(
        [x[:, b * D:(b + 1) * D] for b in range(4)], axis=0)


def _clip_body(x_ref, o_ref):
    x = x_ref[...]
    n = jnp.sqrt(jnp.sum(x * x, axis=0, keepdims=True))
    scale = jnp.where(n > LIMIT, LIMIT / jnp.maximum(n, 1e-12), 1.0)
    o_ref[...] = _to_dense(jnp.transpose(x * scale))


def _sc_bank_body(g3d, items_hbm, bg_out, acc, g_buf, idx_buf,
                  tgt_buf, zbuf):
    c = jax.lax.axis_index("c")
    s = jax.lax.axis_index("s")
    base_row = c * HALF_B + s * RPT
    zv = jnp.zeros((16,), jnp.float32)

    def _zrow(j, carry):
        zbuf[j, pl.ds(0, 16)] = zv
        zbuf[j, pl.ds(16, 16)] = zv
        return carry

    jax.lax.fori_loop(0, ZROWS, _zrow, jnp.int32(0))
    for k in range(RPT // ZROWS):
        pltpu.sync_copy(zbuf, acc.at[pl.ds(s * RPT + k * ZROWS, ZROWS)])
    base_p = (s >> 1) * ROW_BLK + (s & 1) * 256
    for j in range(NCHUNK):
        pltpu.sync_copy(
            items_hbm.at[pl.ds(base_p + (j >> 1) * DBLK + (j & 1) * 128,
                               128)],
            idx_buf.at[j])
    lo = c * HALF_B
    for cc in range(NCHUNK):
        for q in range(8):
            rbv = cc * 128 + q * 16 + jax.lax.broadcasted_iota(
                jnp.int32, (16,), 0)
            p = (rbv & 3) * 256 + (rbv >> 2)
            v = plsc.load_gather(idx_buf, [p >> 7, p & 127])
            loc = v & (ROW_BLK - 1)
            rb_t = (v - loc) + ((loc & (DBLK - 1)) << 2) + (loc >> DSH)
            lrb = rb_t - lo
            oob = (lrb < 0) | (lrb >= HALF_B)
            tgt_buf[cc, pl.ds(q * 16, 16)] = jnp.where(oob, HALF_B, lrb)
    plsc.subcore_barrier()
    for h in range(2):
        pltpu.sync_copy(g3d.at[s, h], g_buf)
        for j in range(NCHUNK // 2):
            pltpu.sync_copy(g_buf.at[j],
                            acc.at[tgt_buf.at[h * (NCHUNK // 2) + j]],
                            add=True)
    plsc.subcore_barrier()
    pltpu.sync_copy(acc.at[pl.ds(s * RPT, RPT)],
                    bg_out.at[pl.ds(base_row, RPT)])


def _m1_body(bg_ref, igb_ref, v_ref, t_ref, n2s_ref):
    i = pl.program_id(0)

    @pl.when(i < NBLK)
    def _phase1():
        bk_t = (jnp.transpose(_to_rows(bg_ref[...]))
                + igb_ref[...])
        n2 = jnp.sum(bk_t * bk_t, axis=0, keepdims=True)
        rid = (i * ROW_BLK
               + jax.lax.broadcasted_iota(jnp.int32, (1, ROW_BLK), 1))
        n2s_ref[pl.ds(i, 1), :] = jnp.where(rid < M, n2, 0.0)

    @pl.when(i == NBLK)
    def _phase2():
        x = n2s_ref[...]
        bits = jax.lax.bitcast_convert_type(x, jnp.int32)

        def step(p, u):
            shift = 28 - 4 * p
            dstar = jnp.int32(0)
            for d in range(1, 16):
                cand = jnp.bitwise_or(u, jnp.left_shift(jnp.int32(d), shift))
                ok = jnp.logical_and(
                    cand > 0,
                    jnp.sum((bits >= cand).astype(jnp.int32)) >= K)
                dstar = dstar + ok.astype(jnp.int32)
            return jnp.bitwise_or(u, jnp.left_shift(dstar, shift))

        u = jax.lax.fori_loop(0, 8, step, jnp.int32(0))
        v = jax.lax.bitcast_convert_type(u, jnp.float32)
        tsum = jnp.sum(jnp.where(x >= v, jnp.sqrt(x), 0.0))
        v_ref[...] = jnp.full((1, 1), v, jnp.float32)
        t_ref[...] = jnp.full((1, 1), tsum / K, jnp.float32)


def _m2_body(bg_ref, igb_ref, e_ref, v_ref, t_ref, o_ref):
    bk_t = (jnp.transpose(_to_rows(bg_ref[...]))
            + igb_ref[...])
    e = e_ref[...]
    v = v_ref[0, 0]
    t = t_ref[0, 0]
    n2 = jnp.sum(bk_t * bk_t, axis=0, keepdims=True)
    bn = jnp.sqrt(n2)
    scale = jnp.where(bn > t, t / jnp.maximum(bn, 1e-12), 1.0)
    factor = jnp.where(n2 >= v, scale, 0.0)
    o_ref[...] = e - LR * (bk_t * factor)


def kernel(items_emb, item_grad_bank, items, items_emb_grad):
    f32 = jnp.float32
    emb_t = jnp.transpose(items_emb)
    igb_t = jnp.transpose(item_grad_bank)
    grad_t = jnp.transpose(items_emb_grad)

    g_dense = pl.pallas_call(
        _clip_body,
        grid=(B // ROW_BLK,),
        in_specs=[pl.BlockSpec((D, ROW_BLK), lambda i: (0, i))],
        out_specs=pl.BlockSpec((DBLK, 128), lambda i: (i, 0)),
        out_shape=jax.ShapeDtypeStruct((B // 4, 128), f32),
    )(grad_t)

    g3d = g_dense.reshape(NS, 2, NCHUNK // 2, 128, D)
    mesh = plsc.VectorSubcoreMesh(core_axis_name="c", subcore_axis_name="s")
    bg = pl.kernel(
        _sc_bank_body,
        out_type=jax.ShapeDtypeStruct((MB, D), f32),
        mesh=mesh,
        scratch_types=[
            pltpu.VMEM_SHARED((HALF_B + 8, D), f32),
            pltpu.VMEM((NCHUNK // 2, 128, D), f32),
            pltpu.VMEM((NCHUNK, 128), jnp.int32),
            pltpu.VMEM((NCHUNK, 128), jnp.int32),
            pltpu.VMEM((ZROWS, D), f32),
        ],
        compiler_params=pltpu.CompilerParams(
            use_tc_tiling_on_sc=False, needs_layout_passes=False),
    )(g3d, items)
    bg_dense = bg.reshape(MD, 128)

    v, t = pl.pallas_call(
        _m1_body,
        grid=(NBLK + 1,),
        in_specs=[
            pl.BlockSpec((DBLK, 128),
                         lambda i: (jnp.where(i < NBLK, i, 0), 0)),
            pl.BlockSpec((D, ROW_BLK),
                         lambda i: (0, jnp.where(i < NBLK, i, 0))),
        ],
        out_specs=[pl.BlockSpec((1, 1), lambda i: (0, 0)),
                   pl.BlockSpec((1, 1), lambda i: (0, 0))],
        out_shape=[jax.ShapeDtypeStruct((1, 1), f32),
                   jax.ShapeDtypeStruct((1, 1), f32)],
        scratch_shapes=[pltpu.VMEM((NBLK, ROW_BLK), f32)],
    )(bg_dense, igb_t)

    out_t = pl.pallas_call(
        _m2_body,
        grid=(NBLK,),
        in_specs=[
            pl.BlockSpec((DBLK, 128), lambda i: (i, 0)),
            pl.BlockSpec((D, ROW_BLK), lambda i: (0, i)),
            pl.BlockSpec((D, ROW_BLK), lambda i: (0, i)),
            pl.BlockSpec(memory_space=pltpu.MemorySpace.SMEM),
            pl.BlockSpec(memory_space=pltpu.MemorySpace.SMEM),
        ],
        out_specs=pl.BlockSpec((D, ROW_BLK), lambda i: (0, i)),
        out_shape=jax.ShapeDtypeStruct((D, M), f32),
    )(bg_dense, igb_t, emb_t, v, t)
    return jnp.transpose(out_t)


# speedup vs baseline: 2.7979x; 1.1764x over previous
import jax
import jax.numpy as jnp
from jax.experimental import pallas as pl
from jax.experimental.pallas import tpu as pltpu
from jax.experimental.pallas import tpu_sc as plsc

M = 100000
D = 32
B = 16384
K = 1000
LR = 0.01
LIMIT = 1.0

NC = 2
NS = 16
ROW_BLK = 4096
DBLK = ROW_BLK // 4
DSH = 10
NBLK = (M + ROW_BLK - 1) // ROW_BLK
MB = NBLK * ROW_BLK
MD = MB // 4
NPAD_ROWS = NBLK * 8

HALF_B = MB // NC
RPT = HALF_B // NS
GPT = B // NS
NCHUNK = GPT // 128
ZROWS = RPT // 16


def _to_dense(y):
    return jnp.concatenate(
        [y[b * DBLK:(b + 1) * DBLK, :] for b in range(4)], axis=1)


def _to_rows(x):
    return jnp.concatenate(
        [x[:, b * D:(b + 1) * D] for b in range(4)], axis=0)


def _clip_body(x_ref, o_ref):
    x = x_ref[...]
    n = jnp.sqrt(jnp.sum(x * x, axis=0, keepdims=True))
    scale = jnp.where(n > LIMIT, LIMIT / jnp.maximum(n, 1e-12), 1.0)
    o_ref[...] = _to_dense(jnp.transpose(x * scale))


def _sc_bank_body(g3d, items_hbm, bg_out, acc, g_buf, idx_buf,
                  tgt_buf, zbuf):
    c = jax.lax.axis_index("c")
    s = jax.lax.axis_index("s")
    base_row = c * HALF_B + s * RPT
    zv = jnp.zeros((16,), jnp.float32)

    def _zrow(j, carry):
        zbuf[j, pl.ds(0, 16)] = zv
        zbuf[j, pl.ds(16, 16)] = zv
        return carry

    jax.lax.fori_loop(0, ZROWS, _zrow, jnp.int32(0))
    for k in range(RPT // ZROWS):
        pltpu.sync_copy(zbuf, acc.at[pl.ds(s * RPT + k * ZROWS, ZROWS)])
    base_p = (s >> 2) * ROW_BLK + (s & 3) * 256
    for j in range(NCHUNK):
        pltpu.sync_copy(
            items_hbm.at[pl.ds(base_p + (j >> 1) * DBLK + (j & 1) * 128,
                               128)],
            idx_buf.at[j])
    lo = c * HALF_B
    for cc in range(NCHUNK):
        for q in range(8):
            rbv = cc * 128 + q * 16 + jax.lax.broadcasted_iota(
                jnp.int32, (16,), 0)
            p = (rbv & 3) * 256 + (rbv >> 2)
            v = plsc.load_gather(idx_buf, [p >> 7, p & 127])
            loc = v & (ROW_BLK - 1)
            rb_t = (v - loc) + ((loc & (DBLK - 1)) << 2) + (loc >> DSH)
            lrb = rb_t - lo
            oob = (lrb < 0) | (lrb >= HALF_B)
            tgt_buf[cc, pl.ds(q * 16, 16)] = jnp.where(oob, HALF_B, lrb)
    plsc.subcore_barrier()
    for h in range(2):
        pltpu.sync_copy(g3d.at[s, h], g_buf)
        for j in range(NCHUNK // 2):
            pltpu.sync_copy(g_buf.at[j],
                            acc.at[tgt_buf.at[h * (NCHUNK // 2) + j]],
                            add=True)
    plsc.subcore_barrier()
    pltpu.sync_copy(acc.at[pl.ds(s * RPT, RPT)],
                    bg_out.at[pl.ds(base_row, RPT)])


def _m1_body(bg_ref, igb_ref, v_ref, t_ref, n2s_ref):
    i = pl.program_id(0)

    @pl.when(i < NBLK)
    def _phase1():
        bk_t = (jnp.transpose(_to_rows(bg_ref[...]))
                + igb_ref[...])
        n2 = jnp.sum(bk_t * bk_t, axis=0, keepdims=True)
        rid = (i * ROW_BLK
               + jax.lax.broadcasted_iota(jnp.int32, (1, ROW_BLK), 1))
        n2s_ref[pl.ds(i, 1), :] = jnp.where(rid < M, n2, 0.0)

    @pl.when(i == NBLK)
    def _phase2():
        x = n2s_ref[...]
        bits = jax.lax.bitcast_convert_type(x, jnp.int32)

        def step(p, u):
            shift = 28 - 4 * p
            dstar = jnp.int32(0)
            for d in range(1, 16):
                cand = jnp.bitwise_or(u, jnp.left_shift(jnp.int32(d), shift))
                ok = jnp.logical_and(
                    cand > 0,
                    jnp.sum((bits >= cand).astype(jnp.int32)) >= K)
                dstar = dstar + ok.astype(jnp.int32)
            return jnp.bitwise_or(u, jnp.left_shift(dstar, shift))

        u = jax.lax.fori_loop(0, 8, step, jnp.int32(0))
        v = jax.lax.bitcast_convert_type(u, jnp.float32)
        tsum = jnp.sum(jnp.where(x >= v, jnp.sqrt(x), 0.0))
        v_ref[...] = jnp.full((1, 1), v, jnp.float32)
        t_ref[...] = jnp.full((1, 1), tsum / K, jnp.float32)


def _m2_body(bg_ref, igb_ref, e_ref, v_ref, t_ref, o_ref):
    bk_t = (jnp.transpose(_to_rows(bg_ref[...]))
            + igb_ref[...])
    e = e_ref[...]
    v = v_ref[0, 0]
    t = t_ref[0, 0]
    n2 = jnp.sum(bk_t * bk_t, axis=0, keepdims=True)
    bn = jnp.sqrt(n2)
    scale = jnp.where(bn > t, t / jnp.maximum(bn, 1e-12), 1.0)
    factor = jnp.where(n2 >= v, scale, 0.0)
    o_ref[...] = e - LR * (bk_t * factor)


def kernel(items_emb, item_grad_bank, items, items_emb_grad):
    f32 = jnp.float32
    emb_t = jnp.transpose(items_emb)
    igb_t = jnp.transpose(item_grad_bank)
    grad_t = jnp.transpose(items_emb_grad)

    g_dense = pl.pallas_call(
        _clip_body,
        grid=(B // ROW_BLK,),
        in_specs=[pl.BlockSpec((D, ROW_BLK), lambda i: (0, i))],
        out_specs=pl.BlockSpec((DBLK, 128), lambda i: (i, 0)),
        out_shape=jax.ShapeDtypeStruct((B // 4, 128), f32),
    )(grad_t)

    g3d = g_dense.reshape(NS, 2, NCHUNK // 2, 128, D)
    mesh = plsc.VectorSubcoreMesh(core_axis_name="c", subcore_axis_name="s")
    bg = pl.kernel(
        _sc_bank_body,
        out_type=jax.ShapeDtypeStruct((MB, D), f32),
        mesh=mesh,
        scratch_types=[
            pltpu.VMEM_SHARED((HALF_B + 8, D), f32),
            pltpu.VMEM((NCHUNK // 2, 128, D), f32),
            pltpu.VMEM((NCHUNK, 128), jnp.int32),
            pltpu.VMEM((NCHUNK, 128), jnp.int32),
            pltpu.VMEM((ZROWS, D), f32),
        ],
        compiler_params=pltpu.CompilerParams(
            use_tc_tiling_on_sc=False, needs_layout_passes=False),
    )(g3d, items)
    bg_dense = bg.reshape(MD, 128)

    v, t = pl.pallas_call(
        _m1_body,
        grid=(NBLK + 1,),
        in_specs=[
            pl.BlockSpec((DBLK, 128),
                         lambda i: (jnp.where(i < NBLK, i, 0), 0)),
            pl.BlockSpec((D, ROW_BLK),
                         lambda i: (0, jnp.where(i < NBLK, i, 0))),
        ],
        out_specs=[pl.BlockSpec((1, 1), lambda i: (0, 0)),
                   pl.BlockSpec((1, 1), lambda i: (0, 0))],
        out_shape=[jax.ShapeDtypeStruct((1, 1), f32),
                   jax.ShapeDtypeStruct((1, 1), f32)],
        scratch_shapes=[pltpu.VMEM((NBLK, ROW_BLK), f32)],
    )(bg_dense, igb_t)

    out_t = pl.pallas_call(
        _m2_body,
        grid=(NBLK,),
        in_specs=[
            pl.BlockSpec((DBLK, 128), lambda i: (i, 0)),
            pl.BlockSpec((D, ROW_BLK), lambda i: (0, i)),
            pl.BlockSpec((D, ROW_BLK), lambda i: (0, i)),
            pl.BlockSpec(memory_space=pltpu.MemorySpace.SMEM),
            pl.BlockSpec(memory_space=pltpu.MemorySpace.SMEM),
        ],
        out_specs=pl.BlockSpec((D, ROW_BLK), lambda i: (0, i)),
        out_shape=jax.ShapeDtypeStruct((D, M), f32),
    )(bg_dense, igb_t, emb_t, v, t)
    return jnp.transpose(out_t)


# speedup vs baseline: 2.9388x; 1.0504x over previous
import jax
import jax.numpy as jnp
from jax.experimental import pallas as pl
from jax.experimental.pallas import tpu as pltpu
from jax.experimental.pallas import tpu_sc as plsc

M = 100000
D = 32
B = 16384
K = 1000
LR = 0.01
LIMIT = 1.0

NC = 2
NS = 16
ROW_BLK = 8192
DBLK = ROW_BLK // 4
DSH = 11
NBLK = (M + ROW_BLK - 1) // ROW_BLK
MB = NBLK * ROW_BLK
MD = MB // 4
NPAD_ROWS = NBLK * 8

HALF_B = MB // NC
RPT = HALF_B // NS
GPT = B // NS
NCHUNK = GPT // 128
ZROWS = RPT // 32


def _to_dense(y):
    return jnp.concatenate(
        [y[b * DBLK:(b + 1) * DBLK, :] for b in range(4)], axis=1)


def _to_rows(x):
    return jnp.concatenate(
        [x[:, b * D:(b + 1) * D] for b in range(4)], axis=0)


def _clip_body(x_ref, o_ref):
    x = x_ref[...]
    n = jnp.sqrt(jnp.sum(x * x, axis=0, keepdims=True))
    scale = jnp.where(n > LIMIT, LIMIT / jnp.maximum(n, 1e-12), 1.0)
    o_ref[...] = _to_dense(jnp.transpose(x * scale))


def _sc_bank_body(g3d, items_hbm, bg_out, acc, g_buf, idx_buf,
                  tgt_buf, zbuf):
    c = jax.lax.axis_index("c")
    s = jax.lax.axis_index("s")
    base_row = c * HALF_B + s * RPT
    zv = jnp.zeros((16,), jnp.float32)

    def _zrow(j, carry):
        zbuf[j, pl.ds(0, 16)] = zv
        zbuf[j, pl.ds(16, 16)] = zv
        return carry

    jax.lax.fori_loop(0, ZROWS, _zrow, jnp.int32(0))
    for k in range(RPT // ZROWS):
        pltpu.sync_copy(zbuf, acc.at[pl.ds(s * RPT + k * ZROWS, ZROWS)])
    base_p = (s >> 3) * ROW_BLK + (s & 7) * 256
    for j in range(NCHUNK):
        pltpu.sync_copy(
            items_hbm.at[pl.ds(base_p + (j >> 1) * DBLK + (j & 1) * 128,
                               128)],
            idx_buf.at[j])
    lo = c * HALF_B
    for cc in range(NCHUNK):
        for q in range(8):
            rbv = cc * 128 + q * 16 + jax.lax.broadcasted_iota(
                jnp.int32, (16,), 0)
            p = (rbv & 3) * 256 + (rbv >> 2)
            v = plsc.load_gather(idx_buf, [p >> 7, p & 127])
            loc = v & (ROW_BLK - 1)
            rb_t = (v - loc) + ((loc & (DBLK - 1)) << 2) + (loc >> DSH)
            lrb = rb_t - lo
            oob = (lrb < 0) | (lrb >= HALF_B)
            tgt_buf[cc, pl.ds(q * 16, 16)] = jnp.where(oob, HALF_B, lrb)
    plsc.subcore_barrier()
    for h in range(2):
        pltpu.sync_copy(g3d.at[s, h], g_buf)
        for j in range(NCHUNK // 2):
            pltpu.sync_copy(g_buf.at[j],
                            acc.at[tgt_buf.at[h * (NCHUNK // 2) + j]],
                            add=True)
    plsc.subcore_barrier()
    pltpu.sync_copy(acc.at[pl.ds(s * RPT, RPT)],
                    bg_out.at[pl.ds(base_row, RPT)])


def _m1_body(bg_ref, igb_ref, v_ref, t_ref, n2s_ref):
    i = pl.program_id(0)

    @pl.when(i < NBLK)
    def _phase1():
        bk_t = (jnp.transpose(_to_rows(bg_ref[...]))
                + igb_ref[...])
        n2 = jnp.sum(bk_t * bk_t, axis=0, keepdims=True)
        rid = (i * ROW_BLK
               + jax.lax.broadcasted_iota(jnp.int32, (1, ROW_BLK), 1))
        n2s_ref[pl.ds(i, 1), :] = jnp.where(rid < M, n2, 0.0)

    @pl.when(i == NBLK)
    def _phase2():
        x = n2s_ref[...]
        bits = jax.lax.bitcast_convert_type(x, jnp.int32)

        def step(p, u):
            shift = 28 - 4 * p
            dstar = jnp.int32(0)
            for d in range(1, 16):
                cand = jnp.bitwise_or(u, jnp.left_shift(jnp.int32(d), shift))
                ok = jnp.logical_and(
                    cand > 0,
                    jnp.sum((bits >= cand).astype(jnp.int32)) >= K)
                dstar = dstar + ok.astype(jnp.int32)
            return jnp.bitwise_or(u, jnp.left_shift(dstar, shift))

        u = jax.lax.fori_loop(0, 8, step, jnp.int32(0))
        v = jax.lax.bitcast_convert_type(u, jnp.float32)
        tsum = jnp.sum(jnp.where(x >= v, jnp.sqrt(x), 0.0))
        v_ref[...] = jnp.full((1, 1), v, jnp.float32)
        t_ref[...] = jnp.full((1, 1), tsum / K, jnp.float32)


def _m2_body(bg_ref, igb_ref, e_ref, v_ref, t_ref, o_ref):
    bk_t = (jnp.transpose(_to_rows(bg_ref[...]))
            + igb_ref[...])
    e = e_ref[...]
    v = v_ref[0, 0]
    t = t_ref[0, 0]
    n2 = jnp.sum(bk_t * bk_t, axis=0, keepdims=True)
    bn = jnp.sqrt(n2)
    scale = jnp.where(bn > t, t / jnp.maximum(bn, 1e-12), 1.0)
    factor = jnp.where(n2 >= v, scale, 0.0)
    o_ref[...] = e - LR * (bk_t * factor)


def kernel(items_emb, item_grad_bank, items, items_emb_grad):
    f32 = jnp.float32
    emb_t = jnp.transpose(items_emb)
    igb_t = jnp.transpose(item_grad_bank)
    grad_t = jnp.transpose(items_emb_grad)

    g_dense = pl.pallas_call(
        _clip_body,
        grid=(B // ROW_BLK,),
        in_specs=[pl.BlockSpec((D, ROW_BLK), lambda i: (0, i))],
        out_specs=pl.BlockSpec((DBLK, 128), lambda i: (i, 0)),
        out_shape=jax.ShapeDtypeStruct((B // 4, 128), f32),
    )(grad_t)

    g3d = g_dense.reshape(NS, 2, NCHUNK // 2, 128, D)
    mesh = plsc.VectorSubcoreMesh(core_axis_name="c", subcore_axis_name="s")
    bg = pl.kernel(
        _sc_bank_body,
        out_type=jax.ShapeDtypeStruct((MB, D), f32),
        mesh=mesh,
        scratch_types=[
            pltpu.VMEM_SHARED((HALF_B + 8, D), f32),
            pltpu.VMEM((NCHUNK // 2, 128, D), f32),
            pltpu.VMEM((NCHUNK, 128), jnp.int32),
            pltpu.VMEM((NCHUNK, 128), jnp.int32),
            pltpu.VMEM((ZROWS, D), f32),
        ],
        compiler_params=pltpu.CompilerParams(
            use_tc_tiling_on_sc=False, needs_layout_passes=False),
    )(g3d, items)
    bg_dense = bg.reshape(MD, 128)

    v, t = pl.pallas_call(
        _m1_body,
        grid=(NBLK + 1,),
        in_specs=[
            pl.BlockSpec((DBLK, 128),
                         lambda i: (jnp.where(i < NBLK, i, 0), 0)),
            pl.BlockSpec((D, ROW_BLK),
                         lambda i: (0, jnp.where(i < NBLK, i, 0))),
        ],
        out_specs=[pl.BlockSpec((1, 1), lambda i: (0, 0)),
                   pl.BlockSpec((1, 1), lambda i: (0, 0))],
        out_shape=[jax.ShapeDtypeStruct((1, 1), f32),
                   jax.ShapeDtypeStruct((1, 1), f32)],
        scratch_shapes=[pltpu.VMEM((NBLK, ROW_BLK), f32)],
    )(bg_dense, igb_t)

    out_t = pl.pallas_call(
        _m2_body,
        grid=(NBLK,),
        in_specs=[
            pl.BlockSpec((DBLK, 128), lambda i: (i, 0)),
            pl.BlockSpec((D, ROW_BLK), lambda i: (0, i)),
            pl.BlockSpec((D, ROW_BLK), lambda i: (0, i)),
            pl.BlockSpec(memory_space=pltpu.MemorySpace.SMEM),
            pl.BlockSpec(memory_space=pltpu.MemorySpace.SMEM),
        ],
        out_specs=pl.BlockSpec((D, ROW_BLK), lambda i: (0, i)),
        out_shape=jax.ShapeDtypeStruct((D, M), f32),
    )(bg_dense, igb_t, emb_t, v, t)
    return jnp.transpose(out_t)
